# baseline (speedup 1.0000x reference)
import functools

import jax
import jax.numpy as jnp
from jax import lax
from jax.experimental import pallas as pl
from jax.experimental.pallas import tpu as pltpu
from jax.experimental.pallas import tpu_sc as plsc

_N = 10000
_E = 160000
_D = 256
_H = 128
_GW = 128
_SG = _E // _GW
_NSUB = 16
_NP = 10240
_RS = _NP // _NSUB
_EB = 1000
_NB = 1000
_C = 2
_EC = _E // _C
_GC = _EC // _GW
_EPS = 1e-5

_f32 = jnp.float32
_bf16 = jnp.bfloat16


def _layer_norm(y, g, b):
    mu = jnp.mean(y, axis=-1, keepdims=True)
    d = y - mu
    var = jnp.mean(d * d, axis=-1, keepdims=True)
    return d * lax.rsqrt(var + _EPS) * g + b


def _bf16_bits(x):
    b = lax.bitcast_convert_type(x, jnp.uint32)
    rb = b + jnp.uint32(0x7FFF) + ((b >> 16) & jnp.uint32(1))
    return rb & jnp.uint32(0xFFFF0000)


def _pack_halves(p):
    lo = _bf16_bits(p[:, :_H]) >> 16
    hi = _bf16_bits(p[:, _H:])
    return lax.bitcast_convert_type(lo | hi, jnp.int32)


def _unpack_halves(u_i32):
    u = lax.bitcast_convert_type(u_i32, jnp.uint32)
    xlo = lax.bitcast_convert_type(u << 16, _f32)
    xhi = lax.bitcast_convert_type(u & jnp.uint32(0xFFFF0000), _f32)
    return jnp.concatenate([xlo, xhi], axis=1)


def _proj_body(x_ref, ws_ref, wr_ref, wn_ref, ps_ref, pr_ref, na_ref):
    x = x_ref[...].astype(_bf16)
    ps_ref[...] = _pack_halves(
        jnp.dot(x, ws_ref[...], preferred_element_type=_f32))
    pr_ref[...] = _pack_halves(
        jnp.dot(x, wr_ref[...], preferred_element_type=_f32))
    na_ref[...] = jnp.dot(x, wn_ref[...], preferred_element_type=_f32)


def _proj(nodes, ws, wr, wn):
    full = lambda shp: pl.BlockSpec(shp, lambda i: (0, 0))
    return pl.pallas_call(
        _proj_body,
        grid=(_N // _NB,),
        in_specs=[pl.BlockSpec((_NB, _D), lambda i: (i, 0)),
                  full((_D, _D)), full((_D, _D)), full((_D, _D))],
        out_specs=[pl.BlockSpec((_NB, _H), lambda i: (i, 0)),
                   pl.BlockSpec((_NB, _H), lambda i: (i, 0)),
                   pl.BlockSpec((_NB, _D), lambda i: (i, 0))],
        out_shape=[jax.ShapeDtypeStruct((_N, _H), jnp.int32),
                   jax.ShapeDtypeStruct((_N, _H), jnp.int32),
                   jax.ShapeDtypeStruct((_N, _D), _f32)],
    )(nodes, ws, wr, wn)


def _sc_gather(table, idx2d):
    mesh = plsc.VectorSubcoreMesh(core_axis_name="c", subcore_axis_name="s")
    n_idx = idx2d.shape[1]

    @functools.partial(
        pl.kernel,
        out_type=jax.ShapeDtypeStruct((n_idx, _H), jnp.int32),
        mesh=mesh)
    def gk(t_hbm, i_hbm, o_hbm):
        def body(i_vmem, o_vmem):
            pltpu.sync_copy(t_hbm.at[i_vmem.at[0]], o_vmem)

        pltpu.emit_pipeline(
            body,
            grid=(n_idx // _GW,),
            in_specs=[pl.BlockSpec((1, _GW), lambda i: (0, i))],
            out_specs=[pl.BlockSpec((_GW, _H), lambda i: (i, 0))],
            core_axis_name=("c", "s"),
            dimension_semantics=(pltpu.PARALLEL,),
        )(i_hbm, o_hbm)

    return gk(table, idx2d)


def _edge_body(gs_ref, gr_ref, e_ref, w1_ref, w2_ref, b1_ref, b2_ref,
               g_ref, b_ref, oe_ref, lo_ref, hi_ref):
    e = e_ref[...]
    x = _unpack_halves(gs_ref[...]) + _unpack_halves(gr_ref[...]) + b1_ref[...]
    x = x + jnp.dot(e.astype(_bf16), w1_ref[...], preferred_element_type=_f32)
    h = jnp.maximum(x, 0.0)
    y = jnp.dot(h.astype(_bf16), w2_ref[...], preferred_element_type=_f32)
    ne = _layer_norm(y + b2_ref[...], g_ref[...], b_ref[...])
    oe_ref[...] = ne + e
    lo_ref[...] = ne[:, :_H]
    hi_ref[...] = ne[:, _H:]


def _edge_mlp(g2, e_full, k, w1c, w2e, b1, b2, g, b):
    full = lambda shp: pl.BlockSpec(shp, lambda i: (0, 0))
    off = k * (_EC // _EB)
    return pl.pallas_call(
        _edge_body,
        grid=(_EC // _EB,),
        in_specs=[pl.BlockSpec((_EB, _H), lambda i: (i, 0)),
                  pl.BlockSpec((_EB, _H), lambda i: (i + _EC // _EB, 0)),
                  pl.BlockSpec((_EB, _D), lambda i: (i + off, 0))] +
                 [full((_D, _D)), full((_D, _D)),
                  full((1, _D)), full((1, _D)), full((1, _D)), full((1, _D))],
        out_specs=[pl.BlockSpec((_EB, _D), lambda i: (i, 0)),
                   pl.BlockSpec((_EB, _H), lambda i: (i, 0)),
                   pl.BlockSpec((_EB, _H), lambda i: (i, 0))],
        out_shape=[jax.ShapeDtypeStruct((_EC, _D), _f32),
                   jax.ShapeDtypeStruct((_EC, _H), _f32),
                   jax.ShapeDtypeStruct((_EC, _H), _f32)],
    )(g2, g2, e_full, w1c, w2e, b1, b2, g, b)


def _sc_scatter(lo_chunks, hi_chunks, recv3d, init_lo, init_hi, g_base):
    mesh = plsc.VectorSubcoreMesh(core_axis_name="c", subcore_axis_name="s")
    half = jax.ShapeDtypeStruct((_NP, _H), _f32)
    n_ch = len(lo_chunks)
    n_groups = n_ch * _GC
    n_slots = (n_groups + _NSUB - 1) // _NSUB

    @functools.partial(
        pl.kernel,
        out_type=(half, half),
        mesh=mesh,
        scratch_types=[
            pltpu.VMEM((_GW, _H), _f32),
            pltpu.VMEM((_GW, _H), _f32),
            pltpu.VMEM((1, _GW), jnp.int32),
            pltpu.VMEM((1, _GW), jnp.int32),
            pltpu.VMEM_SHARED((_NP, _H), _f32),
            pltpu.SemaphoreType.DMA,
            pltpu.SemaphoreType.DMA,
            pltpu.SemaphoreType.DMA,
            pltpu.SemaphoreType.DMA,
        ])
    def sk(*refs):
        (lo_hbm, hi_hbm) = refs[:n_ch], refs[n_ch:2 * n_ch]
        (r_hbm, zlo_hbm, zhi_hbm, olo_hbm, ohi_hbm,
         rows0, rows1, idx0, idx1, acc_sh,
         semr0, semr1, semi0, semi1) = refs[2 * n_ch:]
        c = lax.axis_index("c")
        s = lax.axis_index("s")
        sl = pl.ds(s * _RS, _RS)

        @pl.when(c == 0)
        def _():
            pltpu.sync_copy(zlo_hbm.at[sl], acc_sh.at[sl])

        @pl.when(c == 1)
        def _():
            pltpu.sync_copy(zhi_hbm.at[sl], acc_sh.at[sl])

        plsc.subcore_barrier()

        rows = (rows0, rows1)
        idxs = (idx0, idx1)
        semr = (semr0, semr1)
        semi = (semi0, semi1)

        def scatter_from(src_chunks):
            def for_chunk(gl, fn):
                for kk in range(n_ch):
                    @pl.when((gl >= kk * _GC) & (gl < (kk + 1) * _GC))
                    def _(kk=kk):
                        fn(src_chunks[kk], gl - kk * _GC)

            def issue(t, b):
                gl = t * _NSUB + s

                @pl.when(gl < n_groups)
                def _():
                    pltpu.async_copy(r_hbm.at[g_base + gl], idxs[b], semi[b])

                    def ld(src, off):
                        pltpu.async_copy(src.at[pl.ds(off * _GW, _GW)],
                                         rows[b], semr[b])
                    for_chunk(gl, ld)

            def drain_add(t, b):
                gl = t * _NSUB + s

                @pl.when(gl < n_groups)
                def _():
                    pltpu.make_async_copy(r_hbm.at[g_base + gl],
                                          idxs[b], semi[b]).wait()

                    def wt(src, off):
                        pltpu.make_async_copy(src.at[pl.ds(off * _GW, _GW)],
                                              rows[b], semr[b]).wait()
                    for_chunk(gl, wt)
                    pltpu.sync_copy(rows[b], acc_sh.at[idxs[b].at[0]], add=True)

            issue(0, 0)
            issue(1, 1)

            @pl.loop(0, (n_slots + 1) // 2)
            def _(i):
                t = 2 * i
                drain_add(t, 0)
                issue(t + 2, 0)
                drain_add(t + 1, 1)
                issue(t + 3, 1)

        @pl.when(c == 0)
        def _():
            scatter_from(lo_hbm)

        @pl.when(c == 1)
        def _():
            scatter_from(hi_hbm)

        plsc.subcore_barrier()

        @pl.when(c == 0)
        def _():
            pltpu.sync_copy(acc_sh.at[sl], olo_hbm.at[sl])

        @pl.when(c == 1)
        def _():
            pltpu.sync_copy(acc_sh.at[sl], ohi_hbm.at[sl])

    return sk(*lo_chunks, *hi_chunks, recv3d, init_lo, init_hi)


def _node_body(n_ref, na_ref, alo_ref, ahi_ref, wlo_ref, whi_ref, w2_ref,
               b1_ref, b2_ref, g_ref, b_ref, out_ref):
    x = na_ref[...] + b1_ref[...]
    x = x + jnp.dot(alo_ref[...].astype(_bf16), wlo_ref[...],
                    preferred_element_type=_f32)
    x = x + jnp.dot(ahi_ref[...].astype(_bf16), whi_ref[...],
                    preferred_element_type=_f32)
    h = jnp.maximum(x, 0.0)
    y = jnp.dot(h.astype(_bf16), w2_ref[...], preferred_element_type=_f32)
    nn = _layer_norm(y + b2_ref[...], g_ref[...], b_ref[...])
    out_ref[...] = nn + n_ref[...]


def _node_mlp(nodes, na, alo, ahi, wlo, whi, w2n, b1, b2, g, b):
    full = lambda shp: pl.BlockSpec(shp, lambda i: (0, 0))
    return pl.pallas_call(
        _node_body,
        grid=(_N // _NB,),
        in_specs=[pl.BlockSpec((_NB, _D), lambda i: (i, 0)),
                  pl.BlockSpec((_NB, _D), lambda i: (i, 0)),
                  pl.BlockSpec((_NB, _H), lambda i: (i, 0)),
                  pl.BlockSpec((_NB, _H), lambda i: (i, 0)),
                  full((_H, _D)), full((_H, _D)), full((_D, _D)),
                  full((1, _D)), full((1, _D)), full((1, _D)), full((1, _D))],
        out_specs=pl.BlockSpec((_NB, _D), lambda i: (i, 0)),
        out_shape=jax.ShapeDtypeStruct((_N, _D), _f32),
    )(nodes, na, alo, ahi, wlo, whi, w2n, b1, b2, g, b)


def kernel(node_features, edge_features, senders, receivers,
           W1e, b1e, W2e, b2e, ge, be,
           W1n, b1n, W2n, b2n, gn, bn):
    ws = W1e[:_D].astype(_bf16)
    wr = W1e[_D:2 * _D].astype(_bf16)
    w1c = W1e[2 * _D:].astype(_bf16)
    w2e = W2e.astype(_bf16)
    wna = W1n[:_D].astype(_bf16)
    wlo = W1n[_D:_D + _H].astype(_bf16)
    whi = W1n[_D + _H:].astype(_bf16)
    w2n = W2n.astype(_bf16)
    row = lambda v: v.reshape(1, _D)

    ps, pr, na = _proj(node_features, ws, wr, wna)
    ptab = jnp.concatenate([ps, pr], axis=0)
    idx_arr = jnp.concatenate(
        [senders.reshape(_C, 1, _EC),
         (receivers + jnp.int32(_N)).reshape(_C, 1, _EC)],
        axis=1).reshape(_C, 2 * _EC)
    recv3d = receivers.reshape(_SG, 1, _GW)

    oe, lo, hi = [], [], []
    for k in range(_C):
        g2 = _sc_gather(ptab, idx_arr[k:k + 1])
        o, l, h = _edge_mlp(g2, edge_features, k, w1c, w2e,
                            row(b1e), row(b2e), row(ge), row(be))
        oe.append(o)
        lo.append(l)
        hi.append(h)
    out_edges = jnp.concatenate(oe, axis=0)

    zeros = jnp.zeros((_NP, _H), _f32)
    n1 = 1
    plo, phi = _sc_scatter(lo[:n1], hi[:n1], recv3d, zeros, zeros, 0)
    agg_lo, agg_hi = _sc_scatter(lo[n1:], hi[n1:], recv3d, plo, phi,
                                 n1 * _GC)
    out_nodes = _node_mlp(
        node_features, na, agg_lo[:_N], agg_hi[:_N], wlo, whi, w2n,
        row(b1n), row(b2n), row(gn), row(bn))
    return (out_nodes, out_edges)


# speedup vs baseline: 1.2061x; 1.2061x over previous
import functools

import jax
import jax.numpy as jnp
from jax import lax
from jax.experimental import pallas as pl
from jax.experimental.pallas import tpu as pltpu
from jax.experimental.pallas import tpu_sc as plsc

_N = 10000
_E = 160000
_D = 256
_H = 128
_GW = 128
_SG = _E // _GW
_NSUB = 16
_NP = 10240
_RS = _NP // _NSUB
_EB = 1000
_NB = 1000
_C = 5
_EC = _E // _C
_GC = _EC // _GW
_EPS = 1e-5

_f32 = jnp.float32
_bf16 = jnp.bfloat16


def _layer_norm(y, g, b):
    mu = jnp.mean(y, axis=-1, keepdims=True)
    d = y - mu
    var = jnp.mean(d * d, axis=-1, keepdims=True)
    return d * lax.rsqrt(var + _EPS) * g + b


def _bf16_bits(x):
    b = lax.bitcast_convert_type(x, jnp.uint32)
    rb = b + jnp.uint32(0x7FFF) + ((b >> 16) & jnp.uint32(1))
    return rb & jnp.uint32(0xFFFF0000)


def _pack_halves(p):
    lo = _bf16_bits(p[:, :_H]) >> 16
    hi = _bf16_bits(p[:, _H:])
    return lax.bitcast_convert_type(lo | hi, jnp.int32)


def _unpack_halves(u_i32):
    u = lax.bitcast_convert_type(u_i32, jnp.uint32)
    xlo = lax.bitcast_convert_type(u << 16, _f32)
    xhi = lax.bitcast_convert_type(u & jnp.uint32(0xFFFF0000), _f32)
    return jnp.concatenate([xlo, xhi], axis=1)


def _proj_body(x_ref, ws_ref, wr_ref, wn_ref, ps_ref, pr_ref, na_ref):
    x = x_ref[...].astype(_bf16)
    ps_ref[...] = _pack_halves(
        jnp.dot(x, ws_ref[...], preferred_element_type=_f32))
    pr_ref[...] = _pack_halves(
        jnp.dot(x, wr_ref[...], preferred_element_type=_f32))
    na_ref[...] = jnp.dot(x, wn_ref[...], preferred_element_type=_f32)


def _proj(nodes, ws, wr, wn):
    full = lambda shp: pl.BlockSpec(shp, lambda i: (0, 0))
    return pl.pallas_call(
        _proj_body,
        grid=(_N // _NB,),
        in_specs=[pl.BlockSpec((_NB, _D), lambda i: (i, 0)),
                  full((_D, _D)), full((_D, _D)), full((_D, _D))],
        out_specs=[pl.BlockSpec((_NB, _H), lambda i: (i, 0)),
                   pl.BlockSpec((_NB, _H), lambda i: (i, 0)),
                   pl.BlockSpec((_NB, _D), lambda i: (i, 0))],
        out_shape=[jax.ShapeDtypeStruct((_N, _H), jnp.int32),
                   jax.ShapeDtypeStruct((_N, _H), jnp.int32),
                   jax.ShapeDtypeStruct((_N, _D), _f32)],
    )(nodes, ws, wr, wn)


def _sc_gather(table, idx2d):
    mesh = plsc.VectorSubcoreMesh(core_axis_name="c", subcore_axis_name="s")
    n_idx = idx2d.shape[1]

    @functools.partial(
        pl.kernel,
        out_type=jax.ShapeDtypeStruct((n_idx, _H), jnp.int32),
        mesh=mesh)
    def gk(t_hbm, i_hbm, o_hbm):
        def body(i_vmem, o_vmem):
            pltpu.sync_copy(t_hbm.at[i_vmem.at[0]], o_vmem)

        pltpu.emit_pipeline(
            body,
            grid=(n_idx // _GW,),
            in_specs=[pl.BlockSpec((1, _GW), lambda i: (0, i))],
            out_specs=[pl.BlockSpec((_GW, _H), lambda i: (i, 0))],
            core_axis_name=("c", "s"),
            dimension_semantics=(pltpu.PARALLEL,),
        )(i_hbm, o_hbm)

    return gk(table, idx2d)


def _edge_body(gs_ref, gr_ref, e_ref, w1_ref, w2_ref, b1_ref, b2_ref,
               g_ref, b_ref, oe_ref, lo_ref, hi_ref):
    e = e_ref[...]
    x = _unpack_halves(gs_ref[...]) + _unpack_halves(gr_ref[...]) + b1_ref[...]
    x = x + jnp.dot(e.astype(_bf16), w1_ref[...], preferred_element_type=_f32)
    h = jnp.maximum(x, 0.0)
    y = jnp.dot(h.astype(_bf16), w2_ref[...], preferred_element_type=_f32)
    ne = _layer_norm(y + b2_ref[...], g_ref[...], b_ref[...])
    oe_ref[...] = ne + e
    lo_ref[...] = ne[:, :_H]
    hi_ref[...] = ne[:, _H:]


def _edge_body_alias(gs_ref, gr_ref, e_ref, w1_ref, w2_ref, b1_ref, b2_ref,
                     g_ref, b_ref, oebuf_ref, oe_ref, lo_ref, hi_ref):
    del oebuf_ref
    _edge_body(gs_ref, gr_ref, e_ref, w1_ref, w2_ref, b1_ref, b2_ref,
               g_ref, b_ref, oe_ref, lo_ref, hi_ref)


def _edge_mlp(g2, e_full, k, w1c, w2e, b1, b2, g, b, oe_buf):
    full = lambda shp: pl.BlockSpec(shp, lambda i: (0, 0))
    off = k * (_EC // _EB)
    in_specs = [pl.BlockSpec((_EB, _H), lambda i: (i, 0)),
                pl.BlockSpec((_EB, _H), lambda i: (i + _EC // _EB, 0)),
                pl.BlockSpec((_EB, _D), lambda i: (i + off, 0))] + \
               [full((_D, _D)), full((_D, _D)),
                full((1, _D)), full((1, _D)), full((1, _D)), full((1, _D))]
    args = [g2, g2, e_full, w1c, w2e, b1, b2, g, b]
    aliases = {}
    if oe_buf is not None:
        in_specs.append(pl.BlockSpec((8, _D), lambda i: (0, 0)))
        args.append(oe_buf)
        aliases = {9: 0}
    return pl.pallas_call(
        _edge_body if oe_buf is None else _edge_body_alias,
        grid=(_EC // _EB,),
        in_specs=in_specs,
        out_specs=[pl.BlockSpec((_EB, _D), lambda i: (i + off, 0)),
                   pl.BlockSpec((_EB, _H), lambda i: (i, 0)),
                   pl.BlockSpec((_EB, _H), lambda i: (i, 0))],
        out_shape=[jax.ShapeDtypeStruct((_E, _D), _f32),
                   jax.ShapeDtypeStruct((_EC, _H), _f32),
                   jax.ShapeDtypeStruct((_EC, _H), _f32)],
        input_output_aliases=aliases,
    )(*args)


def _sc_scatter(lo_chunks, hi_chunks, recv3d, init_lo, init_hi, g_base):
    mesh = plsc.VectorSubcoreMesh(core_axis_name="c", subcore_axis_name="s")
    half = jax.ShapeDtypeStruct((_NP, _H), _f32)
    n_ch = len(lo_chunks)
    n_groups = n_ch * _GC
    n_slots = (n_groups + _NSUB - 1) // _NSUB

    @functools.partial(
        pl.kernel,
        out_type=(half, half),
        mesh=mesh,
        scratch_types=[
            pltpu.VMEM((_GW, _H), _f32),
            pltpu.VMEM((_GW, _H), _f32),
            pltpu.VMEM((1, _GW), jnp.int32),
            pltpu.VMEM((1, _GW), jnp.int32),
            pltpu.VMEM_SHARED((_NP, _H), _f32),
            pltpu.SemaphoreType.DMA,
            pltpu.SemaphoreType.DMA,
            pltpu.SemaphoreType.DMA,
            pltpu.SemaphoreType.DMA,
        ])
    def sk(*refs):
        (lo_hbm, hi_hbm) = refs[:n_ch], refs[n_ch:2 * n_ch]
        (r_hbm, zlo_hbm, zhi_hbm, olo_hbm, ohi_hbm,
         rows0, rows1, idx0, idx1, acc_sh,
         semr0, semr1, semi0, semi1) = refs[2 * n_ch:]
        c = lax.axis_index("c")
        s = lax.axis_index("s")
        sl = pl.ds(s * _RS, _RS)

        @pl.when(c == 0)
        def _():
            pltpu.sync_copy(zlo_hbm.at[sl], acc_sh.at[sl])

        @pl.when(c == 1)
        def _():
            pltpu.sync_copy(zhi_hbm.at[sl], acc_sh.at[sl])

        plsc.subcore_barrier()

        rows = (rows0, rows1)
        idxs = (idx0, idx1)
        semr = (semr0, semr1)
        semi = (semi0, semi1)

        def scatter_from(src_chunks):
            for kk in range(n_ch):
                src = src_chunks[kk]
                base = g_base + kk * _GC

                def issue(t, b, src=src, base=base):
                    gl = t * _NSUB + s

                    @pl.when(gl < _GC)
                    def _():
                        pltpu.async_copy(r_hbm.at[base + gl], idxs[b], semi[b])
                        pltpu.async_copy(src.at[pl.ds(gl * _GW, _GW)],
                                         rows[b], semr[b])

                def drain_add(t, b, src=src, base=base):
                    gl = t * _NSUB + s

                    @pl.when(gl < _GC)
                    def _():
                        pltpu.make_async_copy(r_hbm.at[base + gl],
                                              idxs[b], semi[b]).wait()
                        pltpu.make_async_copy(src.at[pl.ds(gl * _GW, _GW)],
                                              rows[b], semr[b]).wait()
                        pltpu.sync_copy(rows[b], acc_sh.at[idxs[b].at[0]],
                                        add=True)

                issue(0, 0)
                issue(1, 1)

                nsl = (_GC + _NSUB - 1) // _NSUB

                @pl.loop(0, (nsl + 1) // 2)
                def _(i):
                    t = 2 * i
                    drain_add(t, 0)
                    issue(t + 2, 0)
                    drain_add(t + 1, 1)
                    issue(t + 3, 1)

        @pl.when(c == 0)
        def _():
            scatter_from(lo_hbm)

        @pl.when(c == 1)
        def _():
            scatter_from(hi_hbm)

        plsc.subcore_barrier()

        @pl.when(c == 0)
        def _():
            pltpu.sync_copy(acc_sh.at[sl], olo_hbm.at[sl])

        @pl.when(c == 1)
        def _():
            pltpu.sync_copy(acc_sh.at[sl], ohi_hbm.at[sl])

    return sk(*lo_chunks, *hi_chunks, recv3d, init_lo, init_hi)


def _node_body(n_ref, na_ref, alo_ref, ahi_ref, wlo_ref, whi_ref, w2_ref,
               b1_ref, b2_ref, g_ref, b_ref, out_ref):
    x = na_ref[...] + b1_ref[...]
    x = x + jnp.dot(alo_ref[...].astype(_bf16), wlo_ref[...],
                    preferred_element_type=_f32)
    x = x + jnp.dot(ahi_ref[...].astype(_bf16), whi_ref[...],
                    preferred_element_type=_f32)
    h = jnp.maximum(x, 0.0)
    y = jnp.dot(h.astype(_bf16), w2_ref[...], preferred_element_type=_f32)
    nn = _layer_norm(y + b2_ref[...], g_ref[...], b_ref[...])
    out_ref[...] = nn + n_ref[...]


def _node_mlp(nodes, na, alo, ahi, wlo, whi, w2n, b1, b2, g, b):
    full = lambda shp: pl.BlockSpec(shp, lambda i: (0, 0))
    return pl.pallas_call(
        _node_body,
        grid=(_N // _NB,),
        in_specs=[pl.BlockSpec((_NB, _D), lambda i: (i, 0)),
                  pl.BlockSpec((_NB, _D), lambda i: (i, 0)),
                  pl.BlockSpec((_NB, _H), lambda i: (i, 0)),
                  pl.BlockSpec((_NB, _H), lambda i: (i, 0)),
                  full((_H, _D)), full((_H, _D)), full((_D, _D)),
                  full((1, _D)), full((1, _D)), full((1, _D)), full((1, _D))],
        out_specs=pl.BlockSpec((_NB, _D), lambda i: (i, 0)),
        out_shape=jax.ShapeDtypeStruct((_N, _D), _f32),
    )(nodes, na, alo, ahi, wlo, whi, w2n, b1, b2, g, b)


def kernel(node_features, edge_features, senders, receivers,
           W1e, b1e, W2e, b2e, ge, be,
           W1n, b1n, W2n, b2n, gn, bn):
    ws = W1e[:_D].astype(_bf16)
    wr = W1e[_D:2 * _D].astype(_bf16)
    w1c = W1e[2 * _D:].astype(_bf16)
    w2e = W2e.astype(_bf16)
    wna = W1n[:_D].astype(_bf16)
    wlo = W1n[_D:_D + _H].astype(_bf16)
    whi = W1n[_D + _H:].astype(_bf16)
    w2n = W2n.astype(_bf16)
    row = lambda v: v.reshape(1, _D)

    ps, pr, na = _proj(node_features, ws, wr, wna)
    ptab = jnp.concatenate([ps, pr], axis=0)
    idx_arr = jnp.concatenate(
        [senders.reshape(_C, 1, _EC),
         (receivers + jnp.int32(_N)).reshape(_C, 1, _EC)],
        axis=1).reshape(_C, 2 * _EC)
    recv3d = receivers.reshape(_SG, 1, _GW)

    lo, hi = [], []
    out_edges = None
    for k in range(_C):
        g2 = _sc_gather(ptab, idx_arr[k:k + 1])
        out_edges, l, h = _edge_mlp(g2, edge_features, k, w1c, w2e,
                                    row(b1e), row(b2e), row(ge), row(be),
                                    out_edges)
        lo.append(l)
        hi.append(h)

    zeros = jnp.zeros((_NP, _H), _f32)
    n1 = 3
    plo, phi = _sc_scatter(lo[:n1], hi[:n1], recv3d, zeros, zeros, 0)
    agg_lo, agg_hi = _sc_scatter(lo[n1:], hi[n1:], recv3d, plo, phi,
                                 n1 * _GC)
    out_nodes = _node_mlp(
        node_features, na, agg_lo, agg_hi, wlo, whi, w2n,
        row(b1n), row(b2n), row(gn), row(bn))
    return (out_nodes, out_edges)


# speedup vs baseline: 1.2378x; 1.0263x over previous
import functools

import jax
import jax.numpy as jnp
from jax import lax
from jax.experimental import pallas as pl
from jax.experimental.pallas import tpu as pltpu
from jax.experimental.pallas import tpu_sc as plsc

_N = 10000
_E = 160000
_D = 256
_H = 128
_GW = 128
_SG = _E // _GW
_NSUB = 16
_NP = 10240
_RS = _NP // _NSUB
_EB = 2000
_NB = 1000
_C = 5
_EC = _E // _C
_GC = _EC // _GW
_EPS = 1e-5

_f32 = jnp.float32
_bf16 = jnp.bfloat16


def _layer_norm(y, g, b):
    mu = jnp.mean(y, axis=-1, keepdims=True)
    d = y - mu
    var = jnp.mean(d * d, axis=-1, keepdims=True)
    return d * lax.rsqrt(var + _EPS) * g + b


def _bf16_bits(x):
    b = lax.bitcast_convert_type(x, jnp.uint32)
    rb = b + jnp.uint32(0x7FFF) + ((b >> 16) & jnp.uint32(1))
    return rb & jnp.uint32(0xFFFF0000)


def _pack_halves(p):
    lo = _bf16_bits(p[:, :_H]) >> 16
    hi = _bf16_bits(p[:, _H:])
    return lax.bitcast_convert_type(lo | hi, jnp.int32)


def _unpack_halves(u_i32):
    u = lax.bitcast_convert_type(u_i32, jnp.uint32)
    xlo = lax.bitcast_convert_type(u << 16, _f32)
    xhi = lax.bitcast_convert_type(u & jnp.uint32(0xFFFF0000), _f32)
    return jnp.concatenate([xlo, xhi], axis=1)


def _proj_body(x_ref, ws_ref, wr_ref, wn_ref, ps_ref, pr_ref, na_ref):
    x = x_ref[...].astype(_bf16)
    ps_ref[...] = _pack_halves(
        jnp.dot(x, ws_ref[...], preferred_element_type=_f32))
    pr_ref[...] = _pack_halves(
        jnp.dot(x, wr_ref[...], preferred_element_type=_f32))
    na_ref[...] = jnp.dot(x, wn_ref[...], preferred_element_type=_f32)


def _proj(nodes, ws, wr, wn):
    full = lambda shp: pl.BlockSpec(shp, lambda i: (0, 0))
    return pl.pallas_call(
        _proj_body,
        grid=(_N // _NB,),
        in_specs=[pl.BlockSpec((_NB, _D), lambda i: (i, 0)),
                  full((_D, _D)), full((_D, _D)), full((_D, _D))],
        out_specs=[pl.BlockSpec((_NB, _H), lambda i: (i, 0)),
                   pl.BlockSpec((_NB, _H), lambda i: (i, 0)),
                   pl.BlockSpec((_NB, _D), lambda i: (i, 0))],
        out_shape=[jax.ShapeDtypeStruct((_N, _H), jnp.int32),
                   jax.ShapeDtypeStruct((_N, _H), jnp.int32),
                   jax.ShapeDtypeStruct((_N, _D), _f32)],
    )(nodes, ws, wr, wn)


def _sc_gather(table, idx2d):
    mesh = plsc.VectorSubcoreMesh(core_axis_name="c", subcore_axis_name="s")
    n_idx = idx2d.shape[1]

    @functools.partial(
        pl.kernel,
        out_type=jax.ShapeDtypeStruct((n_idx, _H), jnp.int32),
        mesh=mesh)
    def gk(t_hbm, i_hbm, o_hbm):
        def body(i_vmem, o_vmem):
            pltpu.sync_copy(t_hbm.at[i_vmem.at[0]], o_vmem)

        pltpu.emit_pipeline(
            body,
            grid=(n_idx // _GW,),
            in_specs=[pl.BlockSpec((1, _GW), lambda i: (0, i))],
            out_specs=[pl.BlockSpec((_GW, _H), lambda i: (i, 0))],
            core_axis_name=("c", "s"),
            dimension_semantics=(pltpu.PARALLEL,),
        )(i_hbm, o_hbm)

    return gk(table, idx2d)


def _edge_body(gs_ref, gr_ref, e_ref, w1_ref, w2_ref, b1_ref, b2_ref,
               g_ref, b_ref, oe_ref, lo_ref, hi_ref):
    e = e_ref[...]
    x = _unpack_halves(gs_ref[...]) + _unpack_halves(gr_ref[...]) + b1_ref[...]
    x = x + jnp.dot(e.astype(_bf16), w1_ref[...], preferred_element_type=_f32)
    h = jnp.maximum(x, 0.0)
    y = jnp.dot(h.astype(_bf16), w2_ref[...], preferred_element_type=_f32)
    ne = _layer_norm(y + b2_ref[...], g_ref[...], b_ref[...])
    oe_ref[...] = ne + e
    lo_ref[...] = ne[:, :_H]
    hi_ref[...] = ne[:, _H:]


def _edge_body_alias(gs_ref, gr_ref, e_ref, w1_ref, w2_ref, b1_ref, b2_ref,
                     g_ref, b_ref, oebuf_ref, oe_ref, lo_ref, hi_ref):
    del oebuf_ref
    _edge_body(gs_ref, gr_ref, e_ref, w1_ref, w2_ref, b1_ref, b2_ref,
               g_ref, b_ref, oe_ref, lo_ref, hi_ref)


def _edge_mlp(g2, e_full, k, w1c, w2e, b1, b2, g, b, oe_buf):
    full = lambda shp: pl.BlockSpec(shp, lambda i: (0, 0))
    off = k * (_EC // _EB)
    in_specs = [pl.BlockSpec((_EB, _H), lambda i: (i, 0)),
                pl.BlockSpec((_EB, _H), lambda i: (i + _EC // _EB, 0)),
                pl.BlockSpec((_EB, _D), lambda i: (i + off, 0))] + \
               [full((_D, _D)), full((_D, _D)),
                full((1, _D)), full((1, _D)), full((1, _D)), full((1, _D))]
    args = [g2, g2, e_full, w1c, w2e, b1, b2, g, b]
    aliases = {}
    if oe_buf is not None:
        in_specs.append(pl.BlockSpec((8, _D), lambda i: (0, 0)))
        args.append(oe_buf)
        aliases = {9: 0}
    return pl.pallas_call(
        _edge_body if oe_buf is None else _edge_body_alias,
        grid=(_EC // _EB,),
        in_specs=in_specs,
        out_specs=[pl.BlockSpec((_EB, _D), lambda i: (i + off, 0)),
                   pl.BlockSpec((_EB, _H), lambda i: (i, 0)),
                   pl.BlockSpec((_EB, _H), lambda i: (i, 0))],
        out_shape=[jax.ShapeDtypeStruct((_E, _D), _f32),
                   jax.ShapeDtypeStruct((_EC, _H), _f32),
                   jax.ShapeDtypeStruct((_EC, _H), _f32)],
        input_output_aliases=aliases,
    )(*args)


def _sc_scatter(lo_chunks, hi_chunks, recv3d, init_lo, init_hi, g_base):
    mesh = plsc.VectorSubcoreMesh(core_axis_name="c", subcore_axis_name="s")
    half = jax.ShapeDtypeStruct((_NP, _H), _f32)
    n_ch = len(lo_chunks)
    n_groups = n_ch * _GC
    n_slots = (n_groups + _NSUB - 1) // _NSUB

    @functools.partial(
        pl.kernel,
        out_type=(half, half),
        mesh=mesh,
        scratch_types=[
            pltpu.VMEM((_GW, _H), _f32),
            pltpu.VMEM((_GW, _H), _f32),
            pltpu.VMEM((1, _GW), jnp.int32),
            pltpu.VMEM((1, _GW), jnp.int32),
            pltpu.VMEM_SHARED((_NP, _H), _f32),
            pltpu.SemaphoreType.DMA,
            pltpu.SemaphoreType.DMA,
            pltpu.SemaphoreType.DMA,
            pltpu.SemaphoreType.DMA,
        ])
    def sk(*refs):
        (lo_hbm, hi_hbm) = refs[:n_ch], refs[n_ch:2 * n_ch]
        (r_hbm, zlo_hbm, zhi_hbm, olo_hbm, ohi_hbm,
         rows0, rows1, idx0, idx1, acc_sh,
         semr0, semr1, semi0, semi1) = refs[2 * n_ch:]
        c = lax.axis_index("c")
        s = lax.axis_index("s")
        sl = pl.ds(s * _RS, _RS)

        @pl.when(c == 0)
        def _():
            pltpu.sync_copy(zlo_hbm.at[sl], acc_sh.at[sl])

        @pl.when(c == 1)
        def _():
            pltpu.sync_copy(zhi_hbm.at[sl], acc_sh.at[sl])

        plsc.subcore_barrier()

        rows = (rows0, rows1)
        idxs = (idx0, idx1)
        semr = (semr0, semr1)
        semi = (semi0, semi1)

        def scatter_from(src_chunks):
            for kk in range(n_ch):
                src = src_chunks[kk]
                base = g_base + kk * _GC

                def issue(t, b, src=src, base=base):
                    gl = t * _NSUB + s

                    @pl.when(gl < _GC)
                    def _():
                        pltpu.async_copy(r_hbm.at[base + gl], idxs[b], semi[b])
                        pltpu.async_copy(src.at[pl.ds(gl * _GW, _GW)],
                                         rows[b], semr[b])

                def drain_add(t, b, src=src, base=base):
                    gl = t * _NSUB + s

                    @pl.when(gl < _GC)
                    def _():
                        pltpu.make_async_copy(r_hbm.at[base + gl],
                                              idxs[b], semi[b]).wait()
                        pltpu.make_async_copy(src.at[pl.ds(gl * _GW, _GW)],
                                              rows[b], semr[b]).wait()
                        pltpu.sync_copy(rows[b], acc_sh.at[idxs[b].at[0]],
                                        add=True)

                issue(0, 0)
                issue(1, 1)

                nsl = (_GC + _NSUB - 1) // _NSUB

                @pl.loop(0, (nsl + 1) // 2)
                def _(i):
                    t = 2 * i
                    drain_add(t, 0)
                    issue(t + 2, 0)
                    drain_add(t + 1, 1)
                    issue(t + 3, 1)

        @pl.when(c == 0)
        def _():
            scatter_from(lo_hbm)

        @pl.when(c == 1)
        def _():
            scatter_from(hi_hbm)

        plsc.subcore_barrier()

        @pl.when(c == 0)
        def _():
            pltpu.sync_copy(acc_sh.at[sl], olo_hbm.at[sl])

        @pl.when(c == 1)
        def _():
            pltpu.sync_copy(acc_sh.at[sl], ohi_hbm.at[sl])

    return sk(*lo_chunks, *hi_chunks, recv3d, init_lo, init_hi)


def _node_body(n_ref, na_ref, alo_ref, ahi_ref, wlo_ref, whi_ref, w2_ref,
               b1_ref, b2_ref, g_ref, b_ref, out_ref):
    x = na_ref[...] + b1_ref[...]
    x = x + jnp.dot(alo_ref[...].astype(_bf16), wlo_ref[...],
                    preferred_element_type=_f32)
    x = x + jnp.dot(ahi_ref[...].astype(_bf16), whi_ref[...],
                    preferred_element_type=_f32)
    h = jnp.maximum(x, 0.0)
    y = jnp.dot(h.astype(_bf16), w2_ref[...], preferred_element_type=_f32)
    nn = _layer_norm(y + b2_ref[...], g_ref[...], b_ref[...])
    out_ref[...] = nn + n_ref[...]


def _node_mlp(nodes, na, alo, ahi, wlo, whi, w2n, b1, b2, g, b):
    full = lambda shp: pl.BlockSpec(shp, lambda i: (0, 0))
    return pl.pallas_call(
        _node_body,
        grid=(_N // _NB,),
        in_specs=[pl.BlockSpec((_NB, _D), lambda i: (i, 0)),
                  pl.BlockSpec((_NB, _D), lambda i: (i, 0)),
                  pl.BlockSpec((_NB, _H), lambda i: (i, 0)),
                  pl.BlockSpec((_NB, _H), lambda i: (i, 0)),
                  full((_H, _D)), full((_H, _D)), full((_D, _D)),
                  full((1, _D)), full((1, _D)), full((1, _D)), full((1, _D))],
        out_specs=pl.BlockSpec((_NB, _D), lambda i: (i, 0)),
        out_shape=jax.ShapeDtypeStruct((_N, _D), _f32),
    )(nodes, na, alo, ahi, wlo, whi, w2n, b1, b2, g, b)


def kernel(node_features, edge_features, senders, receivers,
           W1e, b1e, W2e, b2e, ge, be,
           W1n, b1n, W2n, b2n, gn, bn):
    ws = W1e[:_D].astype(_bf16)
    wr = W1e[_D:2 * _D].astype(_bf16)
    w1c = W1e[2 * _D:].astype(_bf16)
    w2e = W2e.astype(_bf16)
    wna = W1n[:_D].astype(_bf16)
    wlo = W1n[_D:_D + _H].astype(_bf16)
    whi = W1n[_D + _H:].astype(_bf16)
    w2n = W2n.astype(_bf16)
    row = lambda v: v.reshape(1, _D)

    ps, pr, na = _proj(node_features, ws, wr, wna)
    ptab = jnp.concatenate([ps, pr], axis=0)
    idx_arr = jnp.concatenate(
        [senders.reshape(_C, 1, _EC),
         (receivers + jnp.int32(_N)).reshape(_C, 1, _EC)],
        axis=1).reshape(_C, 2 * _EC)
    recv3d = receivers.reshape(_SG, 1, _GW)

    lo, hi = [], []
    out_edges = None
    for k in range(_C):
        g2 = _sc_gather(ptab, idx_arr[k:k + 1])
        out_edges, l, h = _edge_mlp(g2, edge_features, k, w1c, w2e,
                                    row(b1e), row(b2e), row(ge), row(be),
                                    out_edges)
        lo.append(l)
        hi.append(h)

    zeros = jnp.zeros((_NP, _H), _f32)
    n1 = 3
    plo, phi = _sc_scatter(lo[:n1], hi[:n1], recv3d, zeros, zeros, 0)
    agg_lo, agg_hi = _sc_scatter(lo[n1:], hi[n1:], recv3d, plo, phi,
                                 n1 * _GC)
    out_nodes = _node_mlp(
        node_features, na, agg_lo, agg_hi, wlo, whi, w2n,
        row(b1n), row(b2n), row(gn), row(bn))
    return (out_nodes, out_edges)


# speedup vs baseline: 1.2404x; 1.0021x over previous
import functools

import jax
import jax.numpy as jnp
from jax import lax
from jax.experimental import pallas as pl
from jax.experimental.pallas import tpu as pltpu
from jax.experimental.pallas import tpu_sc as plsc

_N = 10000
_E = 160000
_D = 256
_H = 128
_GW = 128
_SG = _E // _GW
_NSUB = 16
_NP = 10240
_RS = _NP // _NSUB
_EB = 4000
_NB = 1000
_C = 5
_EC = _E // _C
_GC = _EC // _GW
_EPS = 1e-5

_f32 = jnp.float32
_bf16 = jnp.bfloat16


def _layer_norm(y, g, b):
    mu = jnp.mean(y, axis=-1, keepdims=True)
    d = y - mu
    var = jnp.mean(d * d, axis=-1, keepdims=True)
    return d * lax.rsqrt(var + _EPS) * g + b


def _bf16_bits(x):
    b = lax.bitcast_convert_type(x, jnp.uint32)
    rb = b + jnp.uint32(0x7FFF) + ((b >> 16) & jnp.uint32(1))
    return rb & jnp.uint32(0xFFFF0000)


def _pack_halves(p):
    lo = _bf16_bits(p[:, :_H]) >> 16
    hi = _bf16_bits(p[:, _H:])
    return lax.bitcast_convert_type(lo | hi, jnp.int32)


def _unpack_halves(u_i32):
    u = lax.bitcast_convert_type(u_i32, jnp.uint32)
    xlo = lax.bitcast_convert_type(u << 16, _f32)
    xhi = lax.bitcast_convert_type(u & jnp.uint32(0xFFFF0000), _f32)
    return jnp.concatenate([xlo, xhi], axis=1)


def _proj_body(x_ref, ws_ref, wr_ref, wn_ref, ps_ref, pr_ref, na_ref):
    x = x_ref[...].astype(_bf16)
    ps_ref[...] = _pack_halves(
        jnp.dot(x, ws_ref[...], preferred_element_type=_f32))
    pr_ref[...] = _pack_halves(
        jnp.dot(x, wr_ref[...], preferred_element_type=_f32))
    na_ref[...] = jnp.dot(x, wn_ref[...], preferred_element_type=_f32)


def _proj(nodes, ws, wr, wn):
    full = lambda shp: pl.BlockSpec(shp, lambda i: (0, 0))
    return pl.pallas_call(
        _proj_body,
        grid=(_N // _NB,),
        in_specs=[pl.BlockSpec((_NB, _D), lambda i: (i, 0)),
                  full((_D, _D)), full((_D, _D)), full((_D, _D))],
        out_specs=[pl.BlockSpec((_NB, _H), lambda i: (i, 0)),
                   pl.BlockSpec((_NB, _H), lambda i: (i, 0)),
                   pl.BlockSpec((_NB, _D), lambda i: (i, 0))],
        out_shape=[jax.ShapeDtypeStruct((_N, _H), jnp.int32),
                   jax.ShapeDtypeStruct((_N, _H), jnp.int32),
                   jax.ShapeDtypeStruct((_N, _D), _f32)],
    )(nodes, ws, wr, wn)


def _sc_gather(table, idx2d):
    mesh = plsc.VectorSubcoreMesh(core_axis_name="c", subcore_axis_name="s")
    n_idx = idx2d.shape[1]

    @functools.partial(
        pl.kernel,
        out_type=jax.ShapeDtypeStruct((n_idx, _H), jnp.int32),
        mesh=mesh)
    def gk(t_hbm, i_hbm, o_hbm):
        def body(i_vmem, o_vmem):
            pltpu.sync_copy(t_hbm.at[i_vmem.at[0]], o_vmem)

        pltpu.emit_pipeline(
            body,
            grid=(n_idx // _GW,),
            in_specs=[pl.BlockSpec((1, _GW), lambda i: (0, i))],
            out_specs=[pl.BlockSpec((_GW, _H), lambda i: (i, 0))],
            core_axis_name=("c", "s"),
            dimension_semantics=(pltpu.PARALLEL,),
        )(i_hbm, o_hbm)

    return gk(table, idx2d)


def _edge_body(gs_ref, gr_ref, e_ref, w1_ref, w2_ref, b1_ref, b2_ref,
               g_ref, b_ref, oe_ref, lo_ref, hi_ref):
    e = e_ref[...]
    x = _unpack_halves(gs_ref[...]) + _unpack_halves(gr_ref[...]) + b1_ref[...]
    x = x + jnp.dot(e.astype(_bf16), w1_ref[...], preferred_element_type=_f32)
    h = jnp.maximum(x, 0.0)
    y = jnp.dot(h.astype(_bf16), w2_ref[...], preferred_element_type=_f32)
    ne = _layer_norm(y + b2_ref[...], g_ref[...], b_ref[...])
    oe_ref[...] = ne + e
    lo_ref[...] = ne[:, :_H]
    hi_ref[...] = ne[:, _H:]


def _edge_body_alias(gs_ref, gr_ref, e_ref, w1_ref, w2_ref, b1_ref, b2_ref,
                     g_ref, b_ref, oebuf_ref, oe_ref, lo_ref, hi_ref):
    del oebuf_ref
    _edge_body(gs_ref, gr_ref, e_ref, w1_ref, w2_ref, b1_ref, b2_ref,
               g_ref, b_ref, oe_ref, lo_ref, hi_ref)


def _edge_mlp(g2, e_full, k, w1c, w2e, b1, b2, g, b, oe_buf):
    full = lambda shp: pl.BlockSpec(shp, lambda i: (0, 0))
    off = k * (_EC // _EB)
    in_specs = [pl.BlockSpec((_EB, _H), lambda i: (i, 0)),
                pl.BlockSpec((_EB, _H), lambda i: (i + _EC // _EB, 0)),
                pl.BlockSpec((_EB, _D), lambda i: (i + off, 0))] + \
               [full((_D, _D)), full((_D, _D)),
                full((1, _D)), full((1, _D)), full((1, _D)), full((1, _D))]
    args = [g2, g2, e_full, w1c, w2e, b1, b2, g, b]
    aliases = {}
    if oe_buf is not None:
        in_specs.append(pl.BlockSpec((8, _D), lambda i: (0, 0)))
        args.append(oe_buf)
        aliases = {9: 0}
    return pl.pallas_call(
        _edge_body if oe_buf is None else _edge_body_alias,
        grid=(_EC // _EB,),
        in_specs=in_specs,
        out_specs=[pl.BlockSpec((_EB, _D), lambda i: (i + off, 0)),
                   pl.BlockSpec((_EB, _H), lambda i: (i, 0)),
                   pl.BlockSpec((_EB, _H), lambda i: (i, 0))],
        out_shape=[jax.ShapeDtypeStruct((_E, _D), _f32),
                   jax.ShapeDtypeStruct((_EC, _H), _f32),
                   jax.ShapeDtypeStruct((_EC, _H), _f32)],
        input_output_aliases=aliases,
    )(*args)


def _sc_scatter(lo_chunks, hi_chunks, recv3d, init_lo, init_hi, g_base):
    mesh = plsc.VectorSubcoreMesh(core_axis_name="c", subcore_axis_name="s")
    half = jax.ShapeDtypeStruct((_NP, _H), _f32)
    n_ch = len(lo_chunks)
    n_groups = n_ch * _GC
    n_slots = (n_groups + _NSUB - 1) // _NSUB

    @functools.partial(
        pl.kernel,
        out_type=(half, half),
        mesh=mesh,
        scratch_types=[
            pltpu.VMEM((_GW, _H), _f32),
            pltpu.VMEM((_GW, _H), _f32),
            pltpu.VMEM((1, _GW), jnp.int32),
            pltpu.VMEM((1, _GW), jnp.int32),
            pltpu.VMEM_SHARED((_NP, _H), _f32),
            pltpu.SemaphoreType.DMA,
            pltpu.SemaphoreType.DMA,
            pltpu.SemaphoreType.DMA,
            pltpu.SemaphoreType.DMA,
        ])
    def sk(*refs):
        (lo_hbm, hi_hbm) = refs[:n_ch], refs[n_ch:2 * n_ch]
        (r_hbm, zlo_hbm, zhi_hbm, olo_hbm, ohi_hbm,
         rows0, rows1, idx0, idx1, acc_sh,
         semr0, semr1, semi0, semi1) = refs[2 * n_ch:]
        c = lax.axis_index("c")
        s = lax.axis_index("s")
        sl = pl.ds(s * _RS, _RS)

        @pl.when(c == 0)
        def _():
            pltpu.sync_copy(zlo_hbm.at[sl], acc_sh.at[sl])

        @pl.when(c == 1)
        def _():
            pltpu.sync_copy(zhi_hbm.at[sl], acc_sh.at[sl])

        plsc.subcore_barrier()

        rows = (rows0, rows1)
        idxs = (idx0, idx1)
        semr = (semr0, semr1)
        semi = (semi0, semi1)

        def scatter_from(src_chunks):
            for kk in range(n_ch):
                src = src_chunks[kk]
                base = g_base + kk * _GC

                def issue(t, b, src=src, base=base):
                    gl = t * _NSUB + s

                    @pl.when(gl < _GC)
                    def _():
                        pltpu.async_copy(r_hbm.at[base + gl], idxs[b], semi[b])
                        pltpu.async_copy(src.at[pl.ds(gl * _GW, _GW)],
                                         rows[b], semr[b])

                def drain_add(t, b, src=src, base=base):
                    gl = t * _NSUB + s

                    @pl.when(gl < _GC)
                    def _():
                        pltpu.make_async_copy(r_hbm.at[base + gl],
                                              idxs[b], semi[b]).wait()
                        pltpu.make_async_copy(src.at[pl.ds(gl * _GW, _GW)],
                                              rows[b], semr[b]).wait()
                        pltpu.sync_copy(rows[b], acc_sh.at[idxs[b].at[0]],
                                        add=True)

                issue(0, 0)
                issue(1, 1)

                nsl = (_GC + _NSUB - 1) // _NSUB

                @pl.loop(0, (nsl + 1) // 2)
                def _(i):
                    t = 2 * i
                    drain_add(t, 0)
                    issue(t + 2, 0)
                    drain_add(t + 1, 1)
                    issue(t + 3, 1)

        @pl.when(c == 0)
        def _():
            scatter_from(lo_hbm)

        @pl.when(c == 1)
        def _():
            scatter_from(hi_hbm)

        plsc.subcore_barrier()

        @pl.when(c == 0)
        def _():
            pltpu.sync_copy(acc_sh.at[sl], olo_hbm.at[sl])

        @pl.when(c == 1)
        def _():
            pltpu.sync_copy(acc_sh.at[sl], ohi_hbm.at[sl])

    return sk(*lo_chunks, *hi_chunks, recv3d, init_lo, init_hi)


def _node_body(n_ref, na_ref, alo_ref, ahi_ref, wlo_ref, whi_ref, w2_ref,
               b1_ref, b2_ref, g_ref, b_ref, out_ref):
    x = na_ref[...] + b1_ref[...]
    x = x + jnp.dot(alo_ref[...].astype(_bf16), wlo_ref[...],
                    preferred_element_type=_f32)
    x = x + jnp.dot(ahi_ref[...].astype(_bf16), whi_ref[...],
                    preferred_element_type=_f32)
    h = jnp.maximum(x, 0.0)
    y = jnp.dot(h.astype(_bf16), w2_ref[...], preferred_element_type=_f32)
    nn = _layer_norm(y + b2_ref[...], g_ref[...], b_ref[...])
    out_ref[...] = nn + n_ref[...]


def _node_mlp(nodes, na, alo, ahi, wlo, whi, w2n, b1, b2, g, b):
    full = lambda shp: pl.BlockSpec(shp, lambda i: (0, 0))
    return pl.pallas_call(
        _node_body,
        grid=(_N // _NB,),
        in_specs=[pl.BlockSpec((_NB, _D), lambda i: (i, 0)),
                  pl.BlockSpec((_NB, _D), lambda i: (i, 0)),
                  pl.BlockSpec((_NB, _H), lambda i: (i, 0)),
                  pl.BlockSpec((_NB, _H), lambda i: (i, 0)),
                  full((_H, _D)), full((_H, _D)), full((_D, _D)),
                  full((1, _D)), full((1, _D)), full((1, _D)), full((1, _D))],
        out_specs=pl.BlockSpec((_NB, _D), lambda i: (i, 0)),
        out_shape=jax.ShapeDtypeStruct((_N, _D), _f32),
    )(nodes, na, alo, ahi, wlo, whi, w2n, b1, b2, g, b)


def kernel(node_features, edge_features, senders, receivers,
           W1e, b1e, W2e, b2e, ge, be,
           W1n, b1n, W2n, b2n, gn, bn):
    ws = W1e[:_D].astype(_bf16)
    wr = W1e[_D:2 * _D].astype(_bf16)
    w1c = W1e[2 * _D:].astype(_bf16)
    w2e = W2e.astype(_bf16)
    wna = W1n[:_D].astype(_bf16)
    wlo = W1n[_D:_D + _H].astype(_bf16)
    whi = W1n[_D + _H:].astype(_bf16)
    w2n = W2n.astype(_bf16)
    row = lambda v: v.reshape(1, _D)

    ps, pr, na = _proj(node_features, ws, wr, wna)
    ptab = jnp.concatenate([ps, pr], axis=0)
    idx_arr = jnp.concatenate(
        [senders.reshape(_C, 1, _EC),
         (receivers + jnp.int32(_N)).reshape(_C, 1, _EC)],
        axis=1).reshape(_C, 2 * _EC)
    recv3d = receivers.reshape(_SG, 1, _GW)

    lo, hi = [], []
    out_edges = None
    for k in range(_C):
        g2 = _sc_gather(ptab, idx_arr[k:k + 1])
        out_edges, l, h = _edge_mlp(g2, edge_features, k, w1c, w2e,
                                    row(b1e), row(b2e), row(ge), row(be),
                                    out_edges)
        lo.append(l)
        hi.append(h)

    zeros = jnp.zeros((_NP, _H), _f32)
    n1 = 3
    plo, phi = _sc_scatter(lo[:n1], hi[:n1], recv3d, zeros, zeros, 0)
    agg_lo, agg_hi = _sc_scatter(lo[n1:], hi[n1:], recv3d, plo, phi,
                                 n1 * _GC)
    out_nodes = _node_mlp(
        node_features, na, agg_lo, agg_hi, wlo, whi, w2n,
        row(b1n), row(b2n), row(gn), row(bn))
    return (out_nodes, out_edges)
